# baseline (speedup 1.0000x reference)
import jax
import jax.numpy as jnp
from jax import lax
from jax.experimental import pallas as pl
from jax.experimental.pallas import tpu as pltpu

N_DOMAIN = 8
EPS = 1e-05
ROWS = 16384
COLS = 1024
BR = 1024
NB = ROWS // BR


def _onehot_t(y_ref, i):
    yv = y_ref[i]
    ids = lax.broadcasted_iota(jnp.int32, (N_DOMAIN, BR), 0)
    return (ids == yv).astype(jnp.float32)


def _fused_kernel(y_ref, g_ref, b_ref, x_ref, out_ref,
                  xbuf, sums, sumsq, cnt, atab, btab):
    p = pl.program_id(0)
    i = pl.program_id(1)

    @pl.when(p == 0)
    def _phase0():
        @pl.when(i == 0)
        def _zero():
            sums[...] = jnp.zeros_like(sums)
            sumsq[...] = jnp.zeros_like(sumsq)
            cnt[...] = jnp.zeros_like(cnt)

        xb = x_ref[...]
        oh = _onehot_t(y_ref, i)
        sums[...] += lax.dot_general(
            oh, xb, (((1,), (0,)), ((), ())),
            preferred_element_type=jnp.float32)
        sumsq[...] += lax.dot_general(
            oh, xb * xb, (((1,), (0,)), ((), ())),
            preferred_element_type=jnp.float32)
        cnt[...] += jnp.broadcast_to(
            jnp.sum(oh, axis=1, keepdims=True), cnt.shape)
        xbuf[pl.ds(i * BR, BR), :] = xb.astype(jnp.bfloat16)

    @pl.when(p == 1)
    def _phase1():
        @pl.when(i == 0)
        def _tables():
            c = cnt[:, :1]
            denom = jnp.maximum(c, 1.0)
            mean = sums[...] / denom
            var = jnp.maximum(sumsq[...] / denom - mean * mean, 0.0)
            scale = g_ref[...] * lax.rsqrt(var + EPS)
            multi = c > 1.0
            atab[...] = jnp.where(multi, scale, 1.0)
            btab[...] = jnp.where(multi, b_ref[...] - mean * scale, 0.0)

        oh = _onehot_t(y_ref, i)
        row_a = lax.dot_general(oh, atab[...], (((0,), (0,)), ((), ())),
                                preferred_element_type=jnp.float32)
        row_b = lax.dot_general(oh, btab[...], (((0,), (0,)), ((), ())),
                                preferred_element_type=jnp.float32)
        xb = xbuf[pl.ds(i * BR, BR), :].astype(jnp.float32)
        out_ref[...] = xb * row_a + row_b


@jax.jit
def kernel(x, y, gamma, beta):
    y3 = y.astype(jnp.int32).reshape(NB, 1, BR)
    out = pl.pallas_call(
        _fused_kernel,
        grid=(2, NB),
        in_specs=[
            pl.BlockSpec((NB, 1, BR), lambda p, i: (0, 0, 0)),
            pl.BlockSpec((1, COLS), lambda p, i: (0, 0)),
            pl.BlockSpec((1, COLS), lambda p, i: (0, 0)),
            pl.BlockSpec((BR, COLS),
                         lambda p, i: (jnp.where(p == 0, i, NB - 1), 0)),
        ],
        out_specs=pl.BlockSpec((BR, COLS), lambda p, i: (i * p, 0)),
        out_shape=jax.ShapeDtypeStruct((ROWS, COLS), jnp.float32),
        scratch_shapes=[
            pltpu.VMEM((ROWS, COLS), jnp.bfloat16),
            pltpu.VMEM((N_DOMAIN, COLS), jnp.float32),
            pltpu.VMEM((N_DOMAIN, COLS), jnp.float32),
            pltpu.VMEM((N_DOMAIN, 128), jnp.float32),
            pltpu.VMEM((N_DOMAIN, COLS), jnp.float32),
            pltpu.VMEM((N_DOMAIN, COLS), jnp.float32),
        ],
    )(y3, gamma, beta, x)
    return out
